# Initial kernel scaffold; baseline (speedup 1.0000x reference)
import jax
import jax.numpy as jnp
from jax.experimental import pallas as pl


def kernel(images, predictions):
    raise NotImplementedError("write your pallas kernel here")


# speedup vs baseline: 7.5561x; 7.5561x over previous
import functools

import numpy as np
import jax
import jax.numpy as jnp
from jax.experimental import pallas as pl
from jax.experimental.pallas import tpu as pltpu
from jax.experimental.pallas import tpu_sc as plsc

_NUM_CLASSES = 80
_CONF = 0.05
_IOU_THR = 0.5
_MAX_PER_CLASS = 100
_BOX_VAR = np.array([0.1, 0.1, 0.2, 0.2], np.float32)
_N_ANCHORS = 49104
_N_PAD = 49152
_NUM_CHUNKS = _N_PAD // 128
_NEG = np.float32(-1e30)
_BIG_I = np.int32(2**30)


@functools.lru_cache(maxsize=None)
def _anchors_np(image_h, image_w):
    aspect_ratios = [0.5, 1.0, 2.0]
    scales = [2 ** 0.0, 2 ** (1.0 / 3.0), 2 ** (2.0 / 3.0)]
    areas = [32.0 ** 2, 64.0 ** 2, 128.0 ** 2, 256.0 ** 2, 512.0 ** 2]
    all_anchors = []
    for idx, level in enumerate(range(3, 8)):
        stride = 2 ** level
        fh = int(np.ceil(image_h / stride))
        fw = int(np.ceil(image_w / stride))
        area = areas[idx]
        dims = []
        for ratio in aspect_ratios:
            ah = np.sqrt(area / ratio)
            aw = area / ah
            for s in scales:
                dims.append([aw * s, ah * s])
        dims = np.array(dims, np.float32)
        cx = (np.arange(fw, dtype=np.float32) + 0.5) * stride
        cy = (np.arange(fh, dtype=np.float32) + 0.5) * stride
        cxg, cyg = np.meshgrid(cx, cy)
        centers = np.stack([cxg, cyg], axis=-1).reshape(-1, 1, 2)
        centers = np.tile(centers, (1, 9, 1))
        dims_t = np.tile(dims[None, :, :], (fh * fw, 1, 1))
        anchors = np.concatenate([centers, dims_t], axis=-1).reshape(-1, 4)
        all_anchors.append(anchors)
    return np.concatenate(all_anchors, axis=0)


def _lane_iota(shape):
    return jax.lax.broadcasted_iota(jnp.int32, shape, len(shape) - 1)


def _cmp_gt(vp, ip, v, i):
    return (vp > v) | ((vp == v) & (ip < i))


def _xor_shuffle(x, j):
    lane = _lane_iota(x.shape)
    fwd = jnp.roll(x, -j, axis=-1)
    bwd = jnp.roll(x, j, axis=-1)
    return jnp.where((lane & j) == 0, fwd, bwd)


def _ce(v, i, j, want_max):
    vp = _xor_shuffle(v, j)
    ip = _xor_shuffle(i, j)
    pg = _cmp_gt(vp, ip, v, i)
    take = want_max == pg
    return jnp.where(take, vp, v), jnp.where(take, ip, i)


def _sort_asc(v, i):
    lane = _lane_iota(v.shape)
    k = 2
    while k <= 128:
        j = k // 2
        while j >= 1:
            want_max = jnp.logical_xor((lane & k) == 0, (lane & j) == 0)
            v, i = _ce(v, i, j, want_max)
            j //= 2
        k *= 2
    return v, i


def _merge_desc(tv, ti, cv, ci):
    pg = _cmp_gt(cv, ci, tv, ti)
    zv = jnp.where(pg, cv, tv)
    zi = jnp.where(pg, ci, ti)
    lane = _lane_iota(zv.shape)
    j = 64
    while j >= 1:
        want_max = (lane & j) == 0
        zv, zi = _ce(zv, zi, j, want_max)
        j //= 2
    return zv, zi


def _topk_body(x_ref, tv_ref, ti_ref):
    j = pl.program_id(1)

    @pl.when(j == 0)
    def _():
        tv_ref[0] = jnp.full((_NUM_CLASSES, 128), _NEG, jnp.float32)
        ti_ref[0] = jnp.full((_NUM_CLASSES, 128), _BIG_I, jnp.int32)

    cv = x_ref[0, 0]
    ci = _lane_iota((_NUM_CLASSES, 128)) + j * 128
    cv, ci = _sort_asc(cv, ci)
    tv, ti = _merge_desc(tv_ref[0], ti_ref[0], cv, ci)
    tv_ref[0] = tv
    ti_ref[0] = ti


def _per_class_topk(scores_t):
    n_img = scores_t.shape[0]
    return pl.pallas_call(
        _topk_body,
        grid=(n_img, _NUM_CHUNKS),
        in_specs=[
            pl.BlockSpec((1, 1, _NUM_CLASSES, 128), lambda i, j: (i, j, 0, 0)),
        ],
        out_specs=[
            pl.BlockSpec((1, _NUM_CLASSES, 128), lambda i, j: (i, 0, 0)),
            pl.BlockSpec((1, _NUM_CLASSES, 128), lambda i, j: (i, 0, 0)),
        ],
        out_shape=[
            jax.ShapeDtypeStruct((n_img, _NUM_CLASSES, 128), jnp.float32),
            jax.ShapeDtypeStruct((n_img, _NUM_CLASSES, 128), jnp.int32),
        ],
    )(scores_t)


def _gather_rows_sc(table, indices):
    num_idx = indices.shape[1]
    window = 128
    mesh = plsc.VectorSubcoreMesh(core_axis_name="core",
                                  subcore_axis_name="subcore")

    @pl.kernel(
        out_type=jax.ShapeDtypeStruct((num_idx, table.shape[1]), table.dtype),
        mesh=mesh,
    )
    def _gather_kernel(x_hbm, i_hbm, o_hbm):
        def body(i_vmem, o_vmem):
            pltpu.sync_copy(x_hbm.at[i_vmem.at[0]], o_vmem)

        pltpu.emit_pipeline(
            body,
            grid=(num_idx // window,),
            in_specs=[pl.BlockSpec((1, window), index_map=lambda i: (0, i))],
            out_specs=[pl.BlockSpec((window, table.shape[1]),
                                    index_map=lambda i: (i, 0))],
            core_axis_name="subcore",
            dimension_semantics=(pltpu.PARALLEL,),
        )(i_hbm, o_hbm)

    return _gather_kernel(table, indices)


def _nms_body(tv_ref, x1_ref, y1_ref, x2_ref, y2_ref, out_ref, iou_ref,
              keep_ref):
    tv = tv_ref[0]
    ts = tv[:, :_MAX_PER_CLASS]
    x1 = x1_ref[0]
    y1 = y1_ref[0]
    x2 = x2_ref[0]
    y2 = y2_ref[0]

    area = (x2 - x1) * (y2 - y1)
    ltx = jnp.maximum(x1[:, :, None], x1[:, None, :])
    lty = jnp.maximum(y1[:, :, None], y1[:, None, :])
    rbx = jnp.minimum(x2[:, :, None], x2[:, None, :])
    rby = jnp.minimum(y2[:, :, None], y2[:, None, :])
    w = jnp.clip(rbx - ltx, 0.0)
    h = jnp.clip(rby - lty, 0.0)
    inter = w * h
    union = area[:, :, None] + area[:, None, :] - inter
    iou_ref[...] = inter / jnp.maximum(union, 1e-8)
    keep_ref[...] = jnp.ones((_NUM_CLASSES, _MAX_PER_CLASS), jnp.float32)

    lane100 = jax.lax.broadcasted_iota(jnp.int32, (_NUM_CLASSES, _MAX_PER_CLASS), 1)

    def nms_step(i, carry):
        iou_i = iou_ref[:, pl.ds(i, 1), :].reshape(_NUM_CLASSES, _MAX_PER_CLASS)
        keep = keep_ref[...]
        keep_i = jnp.sum(keep * (lane100 == i).astype(jnp.float32), axis=1,
                         keepdims=True)
        supp = ((iou_i > _IOU_THR) & (lane100 > i)).astype(jnp.float32)
        keep_ref[...] = keep * (1.0 - keep_i * supp)
        return carry

    jax.lax.fori_loop(0, _MAX_PER_CLASS, nms_step, 0)
    keep = keep_ref[...] * (ts > _CONF).astype(jnp.float32)
    sel = jnp.where(keep > 0.5, ts, -1.0)

    selp = jnp.concatenate(
        [sel, jnp.full((_NUM_CLASSES, 28), _NEG, jnp.float32)], axis=1)
    lane128 = _lane_iota((_NUM_CLASSES, 128))
    row128 = jax.lax.broadcasted_iota(jnp.int32, (_NUM_CLASSES, 128), 0)
    fidx = jnp.where(lane128 < _MAX_PER_CLASS,
                     row128 * _MAX_PER_CLASS + lane128, _BIG_I)
    sv, si = _sort_asc(selp, fidx)
    t2v = jnp.full((1, 128), _NEG, jnp.float32)
    t2i = jnp.full((1, 128), _BIG_I, jnp.int32)
    for r in range(_NUM_CLASSES):
        t2v, t2i = _merge_desc(t2v, t2i, sv[r:r + 1], si[r:r + 1])

    sub128 = jax.lax.broadcasted_iota(jnp.int32, (128, 128), 0)
    lanesq = jax.lax.broadcasted_iota(jnp.int32, (128, 128), 1)
    eye = (sub128 == lanesq).astype(jnp.float32)
    fs_col = jnp.sum(eye * t2v, axis=1, keepdims=True)
    fi_col = jnp.sum(jnp.where(sub128 == lanesq, t2i, 0), axis=1,
                     keepdims=True)

    c_row = t2i // _MAX_PER_CLASS
    k_row = t2i % _MAX_PER_CLASS
    sub80 = jax.lax.broadcasted_iota(jnp.int32, (_NUM_CLASSES, 128), 0)
    sub100 = jax.lax.broadcasted_iota(jnp.int32, (_MAX_PER_CLASS, 128), 0)
    oct_ = (sub80 == c_row).astype(jnp.float32)
    okt = (sub100 == k_row).astype(jnp.float32)

    def pick(coord):
        p = jax.lax.dot(coord, okt, precision=jax.lax.Precision.HIGHEST)
        return jnp.sum(oct_ * p, axis=0, keepdims=True)

    bx1 = jnp.sum(eye * pick(x1), axis=1, keepdims=True)
    by1 = jnp.sum(eye * pick(y1), axis=1, keepdims=True)
    bx2 = jnp.sum(eye * pick(x2), axis=1, keepdims=True)
    by2 = jnp.sum(eye * pick(y2), axis=1, keepdims=True)
    fc_col = (fi_col // _MAX_PER_CLASS).astype(jnp.float32)

    out = jnp.concatenate([bx1, by1, bx2, by2, fs_col, fc_col], axis=1)
    out_ref[0] = out[:_MAX_PER_CLASS, :]


def _nms_finalize(tv, bx1, by1, bx2, by2):
    n_img = tv.shape[0]
    spec_s = pl.BlockSpec((1, _NUM_CLASSES, 128), lambda i: (i, 0, 0))
    spec_b = pl.BlockSpec((1, _NUM_CLASSES, _MAX_PER_CLASS), lambda i: (i, 0, 0))
    return pl.pallas_call(
        _nms_body,
        grid=(n_img,),
        in_specs=[spec_s, spec_b, spec_b, spec_b, spec_b],
        out_specs=pl.BlockSpec((1, _MAX_PER_CLASS, 6), lambda i: (i, 0, 0)),
        out_shape=jax.ShapeDtypeStruct((n_img, _MAX_PER_CLASS, 6), jnp.float32),
        scratch_shapes=[
            pltpu.VMEM((_NUM_CLASSES, _MAX_PER_CLASS, _MAX_PER_CLASS),
                       jnp.float32),
            pltpu.VMEM((_NUM_CLASSES, _MAX_PER_CLASS), jnp.float32),
        ],
    )(tv, bx1, by1, bx2, by2)


def kernel(images, predictions):
    n_img = predictions.shape[0]
    anchors = jnp.asarray(_anchors_np(images.shape[1], images.shape[2]))

    b = predictions[..., :4] * jnp.asarray(_BOX_VAR)
    cxcy = b[..., :2] * anchors[:, 2:] + anchors[:, :2]
    wh = jnp.exp(b[..., 2:]) * anchors[:, 2:]
    boxes = jnp.concatenate([cxcy - 0.5 * wh, cxcy + 0.5 * wh], axis=-1)
    scores = jax.nn.sigmoid(predictions[..., 4:])

    sp = jnp.pad(scores, ((0, 0), (0, _N_PAD - _N_ANCHORS), (0, 0)),
                 constant_values=_NEG)
    st = sp.reshape(n_img, _NUM_CHUNKS, 128, _NUM_CLASSES).transpose(0, 1, 3, 2)
    tv, ti = _per_class_topk(st)

    idx100 = ti[:, :, :_MAX_PER_CLASS]
    img_off = (jnp.arange(n_img, dtype=jnp.int32) * _N_ANCHORS)[:, None, None]
    flat_idx = (idx100 + img_off).reshape(1, n_img * _NUM_CLASSES * _MAX_PER_CLASS)
    table = jnp.pad(boxes.reshape(n_img * _N_ANCHORS, 4), ((0, 0), (0, 124)))
    g = _gather_rows_sc(table, flat_idx)
    gb = g[:, :4].reshape(n_img, _NUM_CLASSES, _MAX_PER_CLASS, 4)

    return _nms_finalize(tv, gb[..., 0], gb[..., 1], gb[..., 2], gb[..., 3])


# speedup vs baseline: 19.0778x; 2.5248x over previous
import functools

import numpy as np
import jax
import jax.numpy as jnp
from jax.experimental import pallas as pl
from jax.experimental.pallas import tpu as pltpu
from jax.experimental.pallas import tpu_sc as plsc

_NUM_CLASSES = 80
_CONF = 0.05
_IOU_THR = 0.5
_MAX_PER_CLASS = 100
_BOX_VAR = np.array([0.1, 0.1, 0.2, 0.2], np.float32)
_N_ANCHORS = 49104
_N_PAD = 49152
_NUM_CHUNKS = _N_PAD // 128
_NEG = np.float32(-1e30)
_BIG_I = np.int32(2**30)


@functools.lru_cache(maxsize=None)
def _anchors_np(image_h, image_w):
    aspect_ratios = [0.5, 1.0, 2.0]
    scales = [2 ** 0.0, 2 ** (1.0 / 3.0), 2 ** (2.0 / 3.0)]
    areas = [32.0 ** 2, 64.0 ** 2, 128.0 ** 2, 256.0 ** 2, 512.0 ** 2]
    all_anchors = []
    for idx, level in enumerate(range(3, 8)):
        stride = 2 ** level
        fh = int(np.ceil(image_h / stride))
        fw = int(np.ceil(image_w / stride))
        area = areas[idx]
        dims = []
        for ratio in aspect_ratios:
            ah = np.sqrt(area / ratio)
            aw = area / ah
            for s in scales:
                dims.append([aw * s, ah * s])
        dims = np.array(dims, np.float32)
        cx = (np.arange(fw, dtype=np.float32) + 0.5) * stride
        cy = (np.arange(fh, dtype=np.float32) + 0.5) * stride
        cxg, cyg = np.meshgrid(cx, cy)
        centers = np.stack([cxg, cyg], axis=-1).reshape(-1, 1, 2)
        centers = np.tile(centers, (1, 9, 1))
        dims_t = np.tile(dims[None, :, :], (fh * fw, 1, 1))
        anchors = np.concatenate([centers, dims_t], axis=-1).reshape(-1, 4)
        all_anchors.append(anchors)
    return np.concatenate(all_anchors, axis=0)


def _lane_iota(shape):
    return jax.lax.broadcasted_iota(jnp.int32, shape, len(shape) - 1)


def _cmp_gt(vp, ip, v, i):
    return (vp > v) | ((vp == v) & (ip < i))


def _xor_shuffle(x, j):
    lane = _lane_iota(x.shape)
    fwd = jnp.roll(x, -j, axis=-1)
    bwd = jnp.roll(x, j, axis=-1)
    return jnp.where((lane & j) == 0, fwd, bwd)


def _ce(v, i, j, want_max):
    vp = _xor_shuffle(v, j)
    ip = _xor_shuffle(i, j)
    pg = _cmp_gt(vp, ip, v, i)
    take = want_max == pg
    return jnp.where(take, vp, v), jnp.where(take, ip, i)


def _sort_asc(v, i):
    lane = _lane_iota(v.shape)
    k = 2
    while k <= 128:
        j = k // 2
        while j >= 1:
            want_max = jnp.logical_xor((lane & k) == 0, (lane & j) == 0)
            v, i = _ce(v, i, j, want_max)
            j //= 2
        k *= 2
    return v, i


def _merge_desc(tv, ti, cv, ci):
    pg = _cmp_gt(cv, ci, tv, ti)
    zv = jnp.where(pg, cv, tv)
    zi = jnp.where(pg, ci, ti)
    lane = _lane_iota(zv.shape)
    j = 64
    while j >= 1:
        want_max = (lane & j) == 0
        zv, zi = _ce(zv, zi, j, want_max)
        j //= 2
    return zv, zi


def _fold_body(x_ref, m_ref):
    m_ref[0, 0] = jnp.max(x_ref[0, 0], axis=1, keepdims=True)


def _chunk_maxes(scores_ct):
    n_img = scores_ct.shape[0]
    return pl.pallas_call(
        _fold_body,
        grid=(n_img, _NUM_CLASSES),
        in_specs=[
            pl.BlockSpec((1, 1, _NUM_CHUNKS, 128), lambda i, c: (i, c, 0, 0)),
        ],
        out_specs=pl.BlockSpec((1, 1, _NUM_CHUNKS, 1), lambda i, c: (i, c, 0, 0)),
        out_shape=jax.ShapeDtypeStruct((n_img, _NUM_CLASSES, _NUM_CHUNKS, 1),
                                       jnp.float32),
    )(scores_ct)


def _topk_body(x_ref, tv_ref, ti_ref):
    j = pl.program_id(1)

    @pl.when(j == 0)
    def _():
        tv_ref[0] = jnp.full((_NUM_CLASSES, 128), _NEG, jnp.float32)
        ti_ref[0] = jnp.full((_NUM_CLASSES, 128), _BIG_I, jnp.int32)

    cv = x_ref[0, 0]
    ci = _lane_iota((_NUM_CLASSES, 128)) + j * 128
    cv, ci = _sort_asc(cv, ci)
    tv, ti = _merge_desc(tv_ref[0], ti_ref[0], cv, ci)
    tv_ref[0] = tv
    ti_ref[0] = ti


def _per_class_topk(scores_t):
    n_img, n_chunks = scores_t.shape[:2]
    return pl.pallas_call(
        _topk_body,
        grid=(n_img, n_chunks),
        in_specs=[
            pl.BlockSpec((1, 1, _NUM_CLASSES, 128), lambda i, j: (i, j, 0, 0)),
        ],
        out_specs=[
            pl.BlockSpec((1, _NUM_CLASSES, 128), lambda i, j: (i, 0, 0)),
            pl.BlockSpec((1, _NUM_CLASSES, 128), lambda i, j: (i, 0, 0)),
        ],
        out_shape=[
            jax.ShapeDtypeStruct((n_img, _NUM_CLASSES, 128), jnp.float32),
            jax.ShapeDtypeStruct((n_img, _NUM_CLASSES, 128), jnp.int32),
        ],
    )(scores_t)


def _cand_topk_body(x_ref, q_ref, tv_ref, ti_ref):
    k = pl.program_id(1)

    @pl.when(k == 0)
    def _():
        tv_ref[0] = jnp.full((_NUM_CLASSES, 128), _NEG, jnp.float32)
        ti_ref[0] = jnp.full((_NUM_CLASSES, 128), _BIG_I, jnp.int32)

    cv = x_ref[0, 0]
    q = q_ref[0]
    ci = q * 128 + _lane_iota((_NUM_CLASSES, 128))
    cv, ci = _sort_asc(cv, ci)
    tv, ti = _merge_desc(tv_ref[0], ti_ref[0], cv, ci)
    tv_ref[0] = tv
    ti_ref[0] = ti


def _cand_topk(cands, cids):
    n_img, n_sel = cands.shape[:2]
    return pl.pallas_call(
        _cand_topk_body,
        grid=(n_img, n_sel),
        in_specs=[
            pl.BlockSpec((1, 1, _NUM_CLASSES, 128), lambda i, k: (i, k, 0, 0)),
            pl.BlockSpec((1, _NUM_CLASSES, 1),
                         lambda i, k: (i * n_sel + k, 0, 0)),
        ],
        out_specs=[
            pl.BlockSpec((1, _NUM_CLASSES, 128), lambda i, k: (i, 0, 0)),
            pl.BlockSpec((1, _NUM_CLASSES, 128), lambda i, k: (i, 0, 0)),
        ],
        out_shape=[
            jax.ShapeDtypeStruct((n_img, _NUM_CLASSES, 128), jnp.float32),
            jax.ShapeDtypeStruct((n_img, _NUM_CLASSES, 128), jnp.int32),
        ],
    )(cands, cids)


def _gather_rows_sc(table, indices):
    num_idx = indices.shape[1]
    window = 128
    mesh = plsc.VectorSubcoreMesh(core_axis_name="core",
                                  subcore_axis_name="subcore")

    @pl.kernel(
        out_type=jax.ShapeDtypeStruct((num_idx, table.shape[1]), table.dtype),
        mesh=mesh,
    )
    def _gather_kernel(x_hbm, i_hbm, o_hbm):
        def body(i_vmem, o_vmem):
            pltpu.sync_copy(x_hbm.at[i_vmem.at[0]], o_vmem)

        pltpu.emit_pipeline(
            body,
            grid=(num_idx // window,),
            in_specs=[pl.BlockSpec((1, window), index_map=lambda i: (0, i))],
            out_specs=[pl.BlockSpec((window, table.shape[1]),
                                    index_map=lambda i: (i, 0))],
            core_axis_name="subcore",
            dimension_semantics=(pltpu.PARALLEL,),
        )(i_hbm, o_hbm)

    return _gather_kernel(table, indices)


def _nms_body(tv_ref, x1_ref, y1_ref, x2_ref, y2_ref, out_ref, iou_ref,
              keep_ref):
    tv = tv_ref[0]
    ts = tv[:, :_MAX_PER_CLASS]
    x1 = x1_ref[0]
    y1 = y1_ref[0]
    x2 = x2_ref[0]
    y2 = y2_ref[0]

    area = (x2 - x1) * (y2 - y1)
    ltx = jnp.maximum(x1[:, :, None], x1[:, None, :])
    lty = jnp.maximum(y1[:, :, None], y1[:, None, :])
    rbx = jnp.minimum(x2[:, :, None], x2[:, None, :])
    rby = jnp.minimum(y2[:, :, None], y2[:, None, :])
    w = jnp.clip(rbx - ltx, 0.0)
    h = jnp.clip(rby - lty, 0.0)
    inter = w * h
    union = area[:, :, None] + area[:, None, :] - inter
    iou_ref[...] = inter / jnp.maximum(union, 1e-8)
    keep_ref[...] = jnp.ones((_NUM_CLASSES, _MAX_PER_CLASS), jnp.float32)

    lane100 = jax.lax.broadcasted_iota(jnp.int32, (_NUM_CLASSES, _MAX_PER_CLASS), 1)

    def nms_step(i, carry):
        iou_i = iou_ref[:, pl.ds(i, 1), :].reshape(_NUM_CLASSES, _MAX_PER_CLASS)
        keep = keep_ref[...]
        keep_i = jnp.sum(keep * (lane100 == i).astype(jnp.float32), axis=1,
                         keepdims=True)
        supp = ((iou_i > _IOU_THR) & (lane100 > i)).astype(jnp.float32)
        keep_ref[...] = keep * (1.0 - keep_i * supp)
        return carry

    jax.lax.fori_loop(0, _MAX_PER_CLASS, nms_step, 0)
    keep = keep_ref[...] * (ts > _CONF).astype(jnp.float32)
    sel = jnp.where(keep > 0.5, ts, -1.0)

    selp = jnp.concatenate(
        [sel, jnp.full((_NUM_CLASSES, 28), _NEG, jnp.float32)], axis=1)
    lane128 = _lane_iota((_NUM_CLASSES, 128))
    row128 = jax.lax.broadcasted_iota(jnp.int32, (_NUM_CLASSES, 128), 0)
    fidx = jnp.where(lane128 < _MAX_PER_CLASS,
                     row128 * _MAX_PER_CLASS + lane128, _BIG_I)
    sv, si = _sort_asc(selp, fidx)
    t2v = jnp.full((1, 128), _NEG, jnp.float32)
    t2i = jnp.full((1, 128), _BIG_I, jnp.int32)
    for r in range(_NUM_CLASSES):
        t2v, t2i = _merge_desc(t2v, t2i, sv[r:r + 1], si[r:r + 1])

    sub128 = jax.lax.broadcasted_iota(jnp.int32, (128, 128), 0)
    lanesq = jax.lax.broadcasted_iota(jnp.int32, (128, 128), 1)
    eye = (sub128 == lanesq).astype(jnp.float32)
    fs_col = jnp.sum(eye * t2v, axis=1, keepdims=True)
    fi_col = jnp.sum(jnp.where(sub128 == lanesq, t2i, 0), axis=1,
                     keepdims=True)

    c_row = t2i // _MAX_PER_CLASS
    k_row = t2i % _MAX_PER_CLASS
    sub80 = jax.lax.broadcasted_iota(jnp.int32, (_NUM_CLASSES, 128), 0)
    sub100 = jax.lax.broadcasted_iota(jnp.int32, (_MAX_PER_CLASS, 128), 0)
    oct_ = (sub80 == c_row).astype(jnp.float32)
    okt = (sub100 == k_row).astype(jnp.float32)

    def pick(coord):
        p = jax.lax.dot(coord, okt, precision=jax.lax.Precision.HIGHEST)
        return jnp.sum(oct_ * p, axis=0, keepdims=True)

    bx1 = jnp.sum(eye * pick(x1), axis=1, keepdims=True)
    by1 = jnp.sum(eye * pick(y1), axis=1, keepdims=True)
    bx2 = jnp.sum(eye * pick(x2), axis=1, keepdims=True)
    by2 = jnp.sum(eye * pick(y2), axis=1, keepdims=True)
    fc_col = (fi_col // _MAX_PER_CLASS).astype(jnp.float32)

    out = jnp.concatenate([bx1, by1, bx2, by2, fs_col, fc_col], axis=1)
    out_ref[0] = out[:_MAX_PER_CLASS, :]


def _nms_finalize(tv, bx1, by1, bx2, by2):
    n_img = tv.shape[0]
    spec_s = pl.BlockSpec((1, _NUM_CLASSES, 128), lambda i: (i, 0, 0))
    spec_b = pl.BlockSpec((1, _NUM_CLASSES, _MAX_PER_CLASS), lambda i: (i, 0, 0))
    return pl.pallas_call(
        _nms_body,
        grid=(n_img,),
        in_specs=[spec_s, spec_b, spec_b, spec_b, spec_b],
        out_specs=pl.BlockSpec((1, _MAX_PER_CLASS, 6), lambda i: (i, 0, 0)),
        out_shape=jax.ShapeDtypeStruct((n_img, _MAX_PER_CLASS, 6), jnp.float32),
        scratch_shapes=[
            pltpu.VMEM((_NUM_CLASSES, _MAX_PER_CLASS, _MAX_PER_CLASS),
                       jnp.float32),
            pltpu.VMEM((_NUM_CLASSES, _MAX_PER_CLASS), jnp.float32),
        ],
    )(tv, bx1, by1, bx2, by2)


def kernel(images, predictions):
    n_img = predictions.shape[0]
    anchors = jnp.asarray(_anchors_np(images.shape[1], images.shape[2]))

    b = predictions[..., :4] * jnp.asarray(_BOX_VAR)
    cxcy = b[..., :2] * anchors[:, 2:] + anchors[:, :2]
    wh = jnp.exp(b[..., 2:]) * anchors[:, 2:]
    boxes = jnp.concatenate([cxcy - 0.5 * wh, cxcy + 0.5 * wh], axis=-1)
    scores = jax.nn.sigmoid(predictions[..., 4:])

    sp = jnp.pad(scores, ((0, 0), (0, _N_PAD - _N_ANCHORS), (0, 0)),
                 constant_values=_NEG)
    sct = sp.reshape(n_img, _NUM_CHUNKS, 128, _NUM_CLASSES).transpose(0, 3, 1, 2)

    m = _chunk_maxes(sct).reshape(n_img, _NUM_CLASSES, _NUM_CHUNKS // 128, 128)
    m2 = m.transpose(0, 2, 1, 3)
    _, qi = _per_class_topk(m2)
    cids = qi[:, :, :_MAX_PER_CLASS]

    class_off = (jnp.arange(n_img, dtype=jnp.int32)[:, None, None] * _NUM_CLASSES
                 + jnp.arange(_NUM_CLASSES, dtype=jnp.int32)[None, :, None])
    row_idx = (class_off * _NUM_CHUNKS + cids).reshape(
        1, n_img * _NUM_CLASSES * _MAX_PER_CLASS)
    cand_rows = _gather_rows_sc(
        sct.reshape(n_img * _NUM_CLASSES * _NUM_CHUNKS, 128), row_idx)
    cands = cand_rows.reshape(
        n_img, _NUM_CLASSES, _MAX_PER_CLASS, 128).transpose(0, 2, 1, 3)
    cids_b = cids.transpose(0, 2, 1).reshape(
        n_img * _MAX_PER_CLASS, _NUM_CLASSES, 1)

    tv, ti = _cand_topk(cands, cids_b)

    idx100 = ti[:, :, :_MAX_PER_CLASS]
    img_off = (jnp.arange(n_img, dtype=jnp.int32) * _N_ANCHORS)[:, None, None]
    flat_idx = (idx100 + img_off).reshape(1, n_img * _NUM_CLASSES * _MAX_PER_CLASS)
    table = jnp.pad(boxes.reshape(n_img * _N_ANCHORS, 4), ((0, 0), (0, 124)))
    g = _gather_rows_sc(table, flat_idx)
    gb = g[:, :4].reshape(n_img, _NUM_CLASSES, _MAX_PER_CLASS, 4)

    return _nms_finalize(tv, gb[..., 0], gb[..., 1], gb[..., 2], gb[..., 3])


# speedup vs baseline: 25.5740x; 1.3405x over previous
import functools

import numpy as np
import jax
import jax.numpy as jnp
from jax.experimental import pallas as pl
from jax.experimental.pallas import tpu as pltpu
from jax.experimental.pallas import tpu_sc as plsc

_NUM_CLASSES = 80
_CONF = 0.05
_IOU_THR = 0.5
_MAX_PER_CLASS = 100
_BOX_VAR = np.array([0.1, 0.1, 0.2, 0.2], np.float32)
_N_ANCHORS = 49104
_N_PAD = 49152
_NUM_CHUNKS = _N_PAD // 128
_NEG = np.float32(-1e30)
_BIG_I = np.int32(2**30)


@functools.lru_cache(maxsize=None)
def _anchors_np(image_h, image_w):
    aspect_ratios = [0.5, 1.0, 2.0]
    scales = [2 ** 0.0, 2 ** (1.0 / 3.0), 2 ** (2.0 / 3.0)]
    areas = [32.0 ** 2, 64.0 ** 2, 128.0 ** 2, 256.0 ** 2, 512.0 ** 2]
    all_anchors = []
    for idx, level in enumerate(range(3, 8)):
        stride = 2 ** level
        fh = int(np.ceil(image_h / stride))
        fw = int(np.ceil(image_w / stride))
        area = areas[idx]
        dims = []
        for ratio in aspect_ratios:
            ah = np.sqrt(area / ratio)
            aw = area / ah
            for s in scales:
                dims.append([aw * s, ah * s])
        dims = np.array(dims, np.float32)
        cx = (np.arange(fw, dtype=np.float32) + 0.5) * stride
        cy = (np.arange(fh, dtype=np.float32) + 0.5) * stride
        cxg, cyg = np.meshgrid(cx, cy)
        centers = np.stack([cxg, cyg], axis=-1).reshape(-1, 1, 2)
        centers = np.tile(centers, (1, 9, 1))
        dims_t = np.tile(dims[None, :, :], (fh * fw, 1, 1))
        anchors = np.concatenate([centers, dims_t], axis=-1).reshape(-1, 4)
        all_anchors.append(anchors)
    return np.concatenate(all_anchors, axis=0)


def _lane_iota(shape):
    return jax.lax.broadcasted_iota(jnp.int32, shape, len(shape) - 1)


def _cmp_gt(vp, ip, v, i):
    return (vp > v) | ((vp == v) & (ip < i))


def _xor_shuffle(x, j):
    lane = _lane_iota(x.shape)
    fwd = jnp.roll(x, -j, axis=-1)
    bwd = jnp.roll(x, j, axis=-1)
    return jnp.where((lane & j) == 0, fwd, bwd)


def _ce(v, i, j, want_max):
    vp = _xor_shuffle(v, j)
    ip = _xor_shuffle(i, j)
    pg = _cmp_gt(vp, ip, v, i)
    take = want_max == pg
    return jnp.where(take, vp, v), jnp.where(take, ip, i)


def _sort_asc(v, i, desc_mask=None):
    lane = _lane_iota(v.shape)
    k = 2
    while k <= 128:
        j = k // 2
        while j >= 1:
            want_max = jnp.logical_xor((lane & k) == 0, (lane & j) == 0)
            if desc_mask is not None:
                want_max = jnp.logical_xor(want_max, desc_mask)
            v, i = _ce(v, i, j, want_max)
            j //= 2
        k *= 2
    return v, i


def _merge_desc(tv, ti, cv, ci, desc=True):
    pg = _cmp_gt(cv, ci, tv, ti)
    zv = jnp.where(pg, cv, tv)
    zi = jnp.where(pg, ci, ti)
    lane = _lane_iota(zv.shape)
    j = 64
    while j >= 1:
        want_max = (lane & j) == 0 if desc else (lane & j) != 0
        zv, zi = _ce(zv, zi, j, want_max)
        j //= 2
    return zv, zi


def _fold_body(x_ref, m_ref):
    m_ref[0, 0] = jnp.max(x_ref[0, 0], axis=1, keepdims=True)


def _chunk_maxes(scores_ct):
    n_img = scores_ct.shape[0]
    return pl.pallas_call(
        _fold_body,
        grid=(n_img, _NUM_CLASSES),
        in_specs=[
            pl.BlockSpec((1, 1, _NUM_CHUNKS, 128), lambda i, c: (i, c, 0, 0)),
        ],
        out_specs=pl.BlockSpec((1, 1, _NUM_CHUNKS, 1), lambda i, c: (i, c, 0, 0)),
        out_shape=jax.ShapeDtypeStruct((n_img, _NUM_CLASSES, _NUM_CHUNKS, 1),
                                       jnp.float32),
    )(scores_ct)


def _topk_body(x_ref, tv_ref, ti_ref):
    j = pl.program_id(1)

    @pl.when(j == 0)
    def _():
        tv_ref[0] = jnp.full((_NUM_CLASSES, 128), _NEG, jnp.float32)
        ti_ref[0] = jnp.full((_NUM_CLASSES, 128), _BIG_I, jnp.int32)

    cv = x_ref[0, 0]
    ci = _lane_iota((_NUM_CLASSES, 128)) + j * 128
    cv, ci = _sort_asc(cv, ci)
    tv, ti = _merge_desc(tv_ref[0], ti_ref[0], cv, ci)
    tv_ref[0] = tv
    ti_ref[0] = ti


def _per_class_topk(scores_t):
    n_img, n_chunks = scores_t.shape[:2]
    return pl.pallas_call(
        _topk_body,
        grid=(n_img, n_chunks),
        in_specs=[
            pl.BlockSpec((1, 1, _NUM_CLASSES, 128), lambda i, j: (i, j, 0, 0)),
        ],
        out_specs=[
            pl.BlockSpec((1, _NUM_CLASSES, 128), lambda i, j: (i, 0, 0)),
            pl.BlockSpec((1, _NUM_CLASSES, 128), lambda i, j: (i, 0, 0)),
        ],
        out_shape=[
            jax.ShapeDtypeStruct((n_img, _NUM_CLASSES, 128), jnp.float32),
            jax.ShapeDtypeStruct((n_img, _NUM_CLASSES, 128), jnp.int32),
        ],
    )(scores_t)


def _cand_topk_body(n_img, x_ref, q_ref, tv_ref, ti_ref):
    k = pl.program_id(0)

    @pl.when(k == 0)
    def _():
        tv_ref[...] = jnp.full((n_img, _NUM_CLASSES, 128), _NEG, jnp.float32)
        ti_ref[...] = jnp.full((n_img, _NUM_CLASSES, 128), _BIG_I, jnp.int32)

    cv = x_ref[...]
    q = q_ref[...]
    ci = q * 128 + _lane_iota(cv.shape)
    desc_mask = jax.lax.broadcasted_iota(jnp.int32, cv.shape, 1) == 0
    cv, ci = _sort_asc(cv, ci, desc_mask)
    rv, ri = _merge_desc(cv[:, 0], ci[:, 0], cv[:, 1], ci[:, 1], desc=False)
    tv, ti = _merge_desc(tv_ref[...], ti_ref[...], rv, ri)
    tv_ref[...] = tv
    ti_ref[...] = ti


def _cand_topk(cands, cids):
    n_img, n_sel = cands.shape[:2]
    body = functools.partial(_cand_topk_body, n_img)
    return pl.pallas_call(
        body,
        grid=(n_sel // 2,),
        in_specs=[
            pl.BlockSpec((n_img, 2, _NUM_CLASSES, 128), lambda k: (0, k, 0, 0)),
            pl.BlockSpec((n_img, 2, _NUM_CLASSES, 1), lambda k: (0, k, 0, 0)),
        ],
        out_specs=[
            pl.BlockSpec((n_img, _NUM_CLASSES, 128), lambda k: (0, 0, 0)),
            pl.BlockSpec((n_img, _NUM_CLASSES, 128), lambda k: (0, 0, 0)),
        ],
        out_shape=[
            jax.ShapeDtypeStruct((n_img, _NUM_CLASSES, 128), jnp.float32),
            jax.ShapeDtypeStruct((n_img, _NUM_CLASSES, 128), jnp.int32),
        ],
    )(cands, cids)


def _gather_rows_sc(table, indices):
    num_idx = indices.shape[1]
    window = 128
    mesh = plsc.VectorSubcoreMesh(core_axis_name="core",
                                  subcore_axis_name="subcore")

    @pl.kernel(
        out_type=jax.ShapeDtypeStruct((num_idx, table.shape[1]), table.dtype),
        mesh=mesh,
    )
    def _gather_kernel(x_hbm, i_hbm, o_hbm):
        def body(i_vmem, o_vmem):
            pltpu.sync_copy(x_hbm.at[i_vmem.at[0]], o_vmem)

        pltpu.emit_pipeline(
            body,
            grid=(num_idx // window,),
            in_specs=[pl.BlockSpec((1, window), index_map=lambda i: (0, i))],
            out_specs=[pl.BlockSpec((window, table.shape[1]),
                                    index_map=lambda i: (i, 0))],
            core_axis_name="subcore",
            dimension_semantics=(pltpu.PARALLEL,),
        )(i_hbm, o_hbm)

    return _gather_kernel(table, indices)


def _nms_body(tv_ref, x1_ref, y1_ref, x2_ref, y2_ref, out_ref, iou_ref,
              keep_ref):
    tv = tv_ref[0]
    ts = tv[:, :_MAX_PER_CLASS]
    x1 = x1_ref[0]
    y1 = y1_ref[0]
    x2 = x2_ref[0]
    y2 = y2_ref[0]

    area = (x2 - x1) * (y2 - y1)
    ltx = jnp.maximum(x1[:, :, None], x1[:, None, :])
    lty = jnp.maximum(y1[:, :, None], y1[:, None, :])
    rbx = jnp.minimum(x2[:, :, None], x2[:, None, :])
    rby = jnp.minimum(y2[:, :, None], y2[:, None, :])
    w = jnp.clip(rbx - ltx, 0.0)
    h = jnp.clip(rby - lty, 0.0)
    inter = w * h
    union = area[:, :, None] + area[:, None, :] - inter
    iou_ref[...] = inter / jnp.maximum(union, 1e-8)
    keep_ref[...] = jnp.ones((_NUM_CLASSES, _MAX_PER_CLASS), jnp.float32)

    lane100 = jax.lax.broadcasted_iota(jnp.int32, (_NUM_CLASSES, _MAX_PER_CLASS), 1)

    def nms_step(i, carry):
        iou_i = iou_ref[:, pl.ds(i, 1), :].reshape(_NUM_CLASSES, _MAX_PER_CLASS)
        keep = keep_ref[...]
        keep_i = jnp.sum(keep * (lane100 == i).astype(jnp.float32), axis=1,
                         keepdims=True)
        supp = ((iou_i > _IOU_THR) & (lane100 > i)).astype(jnp.float32)
        keep_ref[...] = keep * (1.0 - keep_i * supp)
        return carry

    jax.lax.fori_loop(0, _MAX_PER_CLASS, nms_step, 0)
    keep = keep_ref[...] * (ts > _CONF).astype(jnp.float32)
    sel = jnp.where(keep > 0.5, ts, -1.0)

    selp = jnp.concatenate(
        [sel, jnp.full((_NUM_CLASSES, 28), _NEG, jnp.float32)], axis=1)
    lane128 = _lane_iota((_NUM_CLASSES, 128))
    row128 = jax.lax.broadcasted_iota(jnp.int32, (_NUM_CLASSES, 128), 0)
    fidx = jnp.where(lane128 < _MAX_PER_CLASS,
                     row128 * _MAX_PER_CLASS + lane128, _BIG_I)
    sv, si = _sort_asc(selp, fidx)
    t2v = jnp.full((1, 128), _NEG, jnp.float32)
    t2i = jnp.full((1, 128), _BIG_I, jnp.int32)
    for r in range(_NUM_CLASSES):
        t2v, t2i = _merge_desc(t2v, t2i, sv[r:r + 1], si[r:r + 1])

    sub128 = jax.lax.broadcasted_iota(jnp.int32, (128, 128), 0)
    lanesq = jax.lax.broadcasted_iota(jnp.int32, (128, 128), 1)
    eye = (sub128 == lanesq).astype(jnp.float32)
    fs_col = jnp.sum(eye * t2v, axis=1, keepdims=True)
    fi_col = jnp.sum(jnp.where(sub128 == lanesq, t2i, 0), axis=1,
                     keepdims=True)

    c_row = t2i // _MAX_PER_CLASS
    k_row = t2i % _MAX_PER_CLASS
    sub80 = jax.lax.broadcasted_iota(jnp.int32, (_NUM_CLASSES, 128), 0)
    sub100 = jax.lax.broadcasted_iota(jnp.int32, (_MAX_PER_CLASS, 128), 0)
    oct_ = (sub80 == c_row).astype(jnp.float32)
    okt = (sub100 == k_row).astype(jnp.float32)

    def pick(coord):
        p = jax.lax.dot(coord, okt, precision=jax.lax.Precision.HIGHEST)
        return jnp.sum(oct_ * p, axis=0, keepdims=True)

    bx1 = jnp.sum(eye * pick(x1), axis=1, keepdims=True)
    by1 = jnp.sum(eye * pick(y1), axis=1, keepdims=True)
    bx2 = jnp.sum(eye * pick(x2), axis=1, keepdims=True)
    by2 = jnp.sum(eye * pick(y2), axis=1, keepdims=True)
    fc_col = (fi_col // _MAX_PER_CLASS).astype(jnp.float32)

    out = jnp.concatenate([bx1, by1, bx2, by2, fs_col, fc_col], axis=1)
    out_ref[0] = out[:_MAX_PER_CLASS, :]


def _nms_finalize(tv, bx1, by1, bx2, by2):
    n_img = tv.shape[0]
    spec_s = pl.BlockSpec((1, _NUM_CLASSES, 128), lambda i: (i, 0, 0))
    spec_b = pl.BlockSpec((1, _NUM_CLASSES, _MAX_PER_CLASS), lambda i: (i, 0, 0))
    return pl.pallas_call(
        _nms_body,
        grid=(n_img,),
        in_specs=[spec_s, spec_b, spec_b, spec_b, spec_b],
        out_specs=pl.BlockSpec((1, _MAX_PER_CLASS, 6), lambda i: (i, 0, 0)),
        out_shape=jax.ShapeDtypeStruct((n_img, _MAX_PER_CLASS, 6), jnp.float32),
        scratch_shapes=[
            pltpu.VMEM((_NUM_CLASSES, _MAX_PER_CLASS, _MAX_PER_CLASS),
                       jnp.float32),
            pltpu.VMEM((_NUM_CLASSES, _MAX_PER_CLASS), jnp.float32),
        ],
    )(tv, bx1, by1, bx2, by2)


def kernel(images, predictions):
    n_img = predictions.shape[0]
    anchors = jnp.asarray(_anchors_np(images.shape[1], images.shape[2]))

    b = predictions[..., :4] * jnp.asarray(_BOX_VAR)
    cxcy = b[..., :2] * anchors[:, 2:] + anchors[:, :2]
    wh = jnp.exp(b[..., 2:]) * anchors[:, 2:]
    boxes = jnp.concatenate([cxcy - 0.5 * wh, cxcy + 0.5 * wh], axis=-1)
    scores = jax.nn.sigmoid(predictions[..., 4:])

    sp = jnp.pad(scores, ((0, 0), (0, _N_PAD - _N_ANCHORS), (0, 0)),
                 constant_values=_NEG)
    sct = sp.reshape(n_img, _NUM_CHUNKS, 128, _NUM_CLASSES).transpose(0, 3, 1, 2)

    m = _chunk_maxes(sct).reshape(n_img, _NUM_CLASSES, _NUM_CHUNKS // 128, 128)
    m2 = m.transpose(0, 2, 1, 3)
    _, qi = _per_class_topk(m2)
    cids = qi[:, :, :_MAX_PER_CLASS]

    class_off = (jnp.arange(n_img, dtype=jnp.int32)[:, None, None] * _NUM_CLASSES
                 + jnp.arange(_NUM_CLASSES, dtype=jnp.int32)[None, :, None])
    row_idx = (class_off * _NUM_CHUNKS + cids).reshape(
        1, n_img * _NUM_CLASSES * _MAX_PER_CLASS)
    cand_rows = _gather_rows_sc(
        sct.reshape(n_img * _NUM_CLASSES * _NUM_CHUNKS, 128), row_idx)
    cands = cand_rows.reshape(
        n_img, _NUM_CLASSES, _MAX_PER_CLASS, 128).transpose(0, 2, 1, 3)
    cids_b = cids.transpose(0, 2, 1)[..., None]

    tv, ti = _cand_topk(cands, cids_b)

    idx100 = ti[:, :, :_MAX_PER_CLASS]
    img_off = (jnp.arange(n_img, dtype=jnp.int32) * _N_ANCHORS)[:, None, None]
    flat_idx = (idx100 + img_off).reshape(1, n_img * _NUM_CLASSES * _MAX_PER_CLASS)
    table = jnp.pad(boxes.reshape(n_img * _N_ANCHORS, 4), ((0, 0), (0, 124)))
    g = _gather_rows_sc(table, flat_idx)
    gb = g[:, :4].reshape(n_img, _NUM_CLASSES, _MAX_PER_CLASS, 4)

    return _nms_finalize(tv, gb[..., 0], gb[..., 1], gb[..., 2], gb[..., 3])


# speedup vs baseline: 26.7340x; 1.0454x over previous
import functools

import numpy as np
import jax
import jax.numpy as jnp
from jax.experimental import pallas as pl
from jax.experimental.pallas import tpu as pltpu
from jax.experimental.pallas import tpu_sc as plsc

_NUM_CLASSES = 80
_CONF = 0.05
_IOU_THR = 0.5
_MAX_PER_CLASS = 100
_BOX_VAR = np.array([0.1, 0.1, 0.2, 0.2], np.float32)
_N_ANCHORS = 49104
_N_PAD = 49152
_NUM_CHUNKS = _N_PAD // 128
_NEG = np.float32(-1e30)
_BIG_I = np.int32(2**30)


@functools.lru_cache(maxsize=None)
def _anchors_np(image_h, image_w):
    aspect_ratios = [0.5, 1.0, 2.0]
    scales = [2 ** 0.0, 2 ** (1.0 / 3.0), 2 ** (2.0 / 3.0)]
    areas = [32.0 ** 2, 64.0 ** 2, 128.0 ** 2, 256.0 ** 2, 512.0 ** 2]
    all_anchors = []
    for idx, level in enumerate(range(3, 8)):
        stride = 2 ** level
        fh = int(np.ceil(image_h / stride))
        fw = int(np.ceil(image_w / stride))
        area = areas[idx]
        dims = []
        for ratio in aspect_ratios:
            ah = np.sqrt(area / ratio)
            aw = area / ah
            for s in scales:
                dims.append([aw * s, ah * s])
        dims = np.array(dims, np.float32)
        cx = (np.arange(fw, dtype=np.float32) + 0.5) * stride
        cy = (np.arange(fh, dtype=np.float32) + 0.5) * stride
        cxg, cyg = np.meshgrid(cx, cy)
        centers = np.stack([cxg, cyg], axis=-1).reshape(-1, 1, 2)
        centers = np.tile(centers, (1, 9, 1))
        dims_t = np.tile(dims[None, :, :], (fh * fw, 1, 1))
        anchors = np.concatenate([centers, dims_t], axis=-1).reshape(-1, 4)
        all_anchors.append(anchors)
    return np.concatenate(all_anchors, axis=0)


def _lane_iota(shape):
    return jax.lax.broadcasted_iota(jnp.int32, shape, len(shape) - 1)


def _cmp_gt(vp, ip, v, i):
    return (vp > v) | ((vp == v) & (ip < i))


def _xor_shuffle(x, j):
    lane = _lane_iota(x.shape)
    fwd = jnp.roll(x, -j, axis=-1)
    bwd = jnp.roll(x, j, axis=-1)
    return jnp.where((lane & j) == 0, fwd, bwd)


def _ce(v, i, j, want_max):
    vp = _xor_shuffle(v, j)
    ip = _xor_shuffle(i, j)
    pg = _cmp_gt(vp, ip, v, i)
    take = want_max == pg
    return jnp.where(take, vp, v), jnp.where(take, ip, i)


def _sort_asc(v, i, desc_mask=None):
    lane = _lane_iota(v.shape)
    k = 2
    while k <= 128:
        j = k // 2
        while j >= 1:
            want_max = jnp.logical_xor((lane & k) == 0, (lane & j) == 0)
            if desc_mask is not None:
                want_max = jnp.logical_xor(want_max, desc_mask)
            v, i = _ce(v, i, j, want_max)
            j //= 2
        k *= 2
    return v, i


def _merge_desc(tv, ti, cv, ci, desc=True):
    pg = _cmp_gt(cv, ci, tv, ti)
    zv = jnp.where(pg, cv, tv)
    zi = jnp.where(pg, ci, ti)
    lane = _lane_iota(zv.shape)
    j = 64
    while j >= 1:
        want_max = (lane & j) == 0 if desc else (lane & j) != 0
        zv, zi = _ce(zv, zi, j, want_max)
        j //= 2
    return zv, zi


def _fold_body(x_ref, m_ref):
    m_ref[0, 0] = jnp.max(x_ref[0, 0], axis=1, keepdims=True)


def _chunk_maxes(scores_ct):
    n_img = scores_ct.shape[0]
    return pl.pallas_call(
        _fold_body,
        grid=(n_img, _NUM_CLASSES),
        in_specs=[
            pl.BlockSpec((1, 1, _NUM_CHUNKS, 128), lambda i, c: (i, c, 0, 0)),
        ],
        out_specs=pl.BlockSpec((1, 1, _NUM_CHUNKS, 1), lambda i, c: (i, c, 0, 0)),
        out_shape=jax.ShapeDtypeStruct((n_img, _NUM_CLASSES, _NUM_CHUNKS, 1),
                                       jnp.float32),
    )(scores_ct)


def _topk_body(x_ref, tv_ref, ti_ref):
    j = pl.program_id(1)

    @pl.when(j == 0)
    def _():
        tv_ref[0] = jnp.full((_NUM_CLASSES, 128), _NEG, jnp.float32)
        ti_ref[0] = jnp.full((_NUM_CLASSES, 128), _BIG_I, jnp.int32)

    cv = x_ref[0, 0]
    ci = _lane_iota((_NUM_CLASSES, 128)) + j * 128
    cv, ci = _sort_asc(cv, ci)
    tv, ti = _merge_desc(tv_ref[0], ti_ref[0], cv, ci)
    tv_ref[0] = tv
    ti_ref[0] = ti


def _per_class_topk(scores_t):
    n_img, n_chunks = scores_t.shape[:2]
    return pl.pallas_call(
        _topk_body,
        grid=(n_img, n_chunks),
        in_specs=[
            pl.BlockSpec((1, 1, _NUM_CLASSES, 128), lambda i, j: (i, j, 0, 0)),
        ],
        out_specs=[
            pl.BlockSpec((1, _NUM_CLASSES, 128), lambda i, j: (i, 0, 0)),
            pl.BlockSpec((1, _NUM_CLASSES, 128), lambda i, j: (i, 0, 0)),
        ],
        out_shape=[
            jax.ShapeDtypeStruct((n_img, _NUM_CLASSES, 128), jnp.float32),
            jax.ShapeDtypeStruct((n_img, _NUM_CLASSES, 128), jnp.int32),
        ],
    )(scores_t)


def _cand_topk_body(n_img, x_ref, q_ref, tv_ref, ti_ref):
    k = pl.program_id(0)

    @pl.when(k == 0)
    def _():
        tv_ref[...] = jnp.full((n_img, _NUM_CLASSES, 128), _NEG, jnp.float32)
        ti_ref[...] = jnp.full((n_img, _NUM_CLASSES, 128), _BIG_I, jnp.int32)

    cv = x_ref[...]
    q = q_ref[...]
    ci = q * 128 + _lane_iota(cv.shape)
    desc_mask = jax.lax.broadcasted_iota(jnp.int32, cv.shape, 1) % 2 == 0
    cv, ci = _sort_asc(cv, ci, desc_mask)
    r0v, r0i = _merge_desc(cv[:, 0], ci[:, 0], cv[:, 1], ci[:, 1], desc=False)
    r1v, r1i = _merge_desc(cv[:, 2], ci[:, 2], cv[:, 3], ci[:, 3], desc=True)
    rv, ri = _merge_desc(r1v, r1i, r0v, r0i, desc=False)
    tv, ti = _merge_desc(tv_ref[...], ti_ref[...], rv, ri)
    tv_ref[...] = tv
    ti_ref[...] = ti


def _cand_topk(cands, cids):
    n_img, n_sel = cands.shape[:2]
    body = functools.partial(_cand_topk_body, n_img)
    return pl.pallas_call(
        body,
        grid=(n_sel // 4,),
        in_specs=[
            pl.BlockSpec((n_img, 4, _NUM_CLASSES, 128), lambda k: (0, k, 0, 0)),
            pl.BlockSpec((n_img, 4, _NUM_CLASSES, 1), lambda k: (0, k, 0, 0)),
        ],
        out_specs=[
            pl.BlockSpec((n_img, _NUM_CLASSES, 128), lambda k: (0, 0, 0)),
            pl.BlockSpec((n_img, _NUM_CLASSES, 128), lambda k: (0, 0, 0)),
        ],
        out_shape=[
            jax.ShapeDtypeStruct((n_img, _NUM_CLASSES, 128), jnp.float32),
            jax.ShapeDtypeStruct((n_img, _NUM_CLASSES, 128), jnp.int32),
        ],
    )(cands, cids)


def _gather_rows_sc(table, indices):
    num_idx = indices.shape[1]
    window = 128
    mesh = plsc.VectorSubcoreMesh(core_axis_name="core",
                                  subcore_axis_name="subcore")

    @pl.kernel(
        out_type=jax.ShapeDtypeStruct((num_idx, table.shape[1]), table.dtype),
        mesh=mesh,
    )
    def _gather_kernel(x_hbm, i_hbm, o_hbm):
        def body(i_vmem, o_vmem):
            pltpu.sync_copy(x_hbm.at[i_vmem.at[0]], o_vmem)

        pltpu.emit_pipeline(
            body,
            grid=(num_idx // window,),
            in_specs=[pl.BlockSpec((1, window), index_map=lambda i: (0, i))],
            out_specs=[pl.BlockSpec((window, table.shape[1]),
                                    index_map=lambda i: (i, 0))],
            core_axis_name="subcore",
            dimension_semantics=(pltpu.PARALLEL,),
        )(i_hbm, o_hbm)

    return _gather_kernel(table, indices)


def _nms_body(tv_ref, x1_ref, y1_ref, x2_ref, y2_ref, out_ref, iou_ref,
              keep_ref):
    tv = tv_ref[0]
    ts = tv[:, :_MAX_PER_CLASS]
    x1 = x1_ref[0]
    y1 = y1_ref[0]
    x2 = x2_ref[0]
    y2 = y2_ref[0]

    area = (x2 - x1) * (y2 - y1)
    ltx = jnp.maximum(x1[:, :, None], x1[:, None, :])
    lty = jnp.maximum(y1[:, :, None], y1[:, None, :])
    rbx = jnp.minimum(x2[:, :, None], x2[:, None, :])
    rby = jnp.minimum(y2[:, :, None], y2[:, None, :])
    w = jnp.clip(rbx - ltx, 0.0)
    h = jnp.clip(rby - lty, 0.0)
    inter = w * h
    union = area[:, :, None] + area[:, None, :] - inter
    iou_ref[...] = inter / jnp.maximum(union, 1e-8)
    keep_ref[...] = jnp.ones((_NUM_CLASSES, _MAX_PER_CLASS), jnp.float32)

    lane100 = jax.lax.broadcasted_iota(jnp.int32, (_NUM_CLASSES, _MAX_PER_CLASS), 1)

    def nms_step(i, carry):
        iou_i = iou_ref[:, pl.ds(i, 1), :].reshape(_NUM_CLASSES, _MAX_PER_CLASS)
        keep = keep_ref[...]
        keep_i = jnp.sum(keep * (lane100 == i).astype(jnp.float32), axis=1,
                         keepdims=True)
        supp = ((iou_i > _IOU_THR) & (lane100 > i)).astype(jnp.float32)
        keep_ref[...] = keep * (1.0 - keep_i * supp)
        return carry

    jax.lax.fori_loop(0, _MAX_PER_CLASS, nms_step, 0)
    keep = keep_ref[...] * (ts > _CONF).astype(jnp.float32)
    sel = jnp.where(keep > 0.5, ts, -1.0)

    selp = jnp.concatenate(
        [sel, jnp.full((_NUM_CLASSES, 28), _NEG, jnp.float32)], axis=1)
    lane128 = _lane_iota((_NUM_CLASSES, 128))
    row128 = jax.lax.broadcasted_iota(jnp.int32, (_NUM_CLASSES, 128), 0)
    fidx = jnp.where(lane128 < _MAX_PER_CLASS,
                     row128 * _MAX_PER_CLASS + lane128, _BIG_I)
    sv, si = _sort_asc(selp, fidx)
    t2v = jnp.full((1, 128), _NEG, jnp.float32)
    t2i = jnp.full((1, 128), _BIG_I, jnp.int32)
    for r in range(_NUM_CLASSES):
        t2v, t2i = _merge_desc(t2v, t2i, sv[r:r + 1], si[r:r + 1])

    sub128 = jax.lax.broadcasted_iota(jnp.int32, (128, 128), 0)
    lanesq = jax.lax.broadcasted_iota(jnp.int32, (128, 128), 1)
    eye = (sub128 == lanesq).astype(jnp.float32)
    fs_col = jnp.sum(eye * t2v, axis=1, keepdims=True)
    fi_col = jnp.sum(jnp.where(sub128 == lanesq, t2i, 0), axis=1,
                     keepdims=True)

    c_row = t2i // _MAX_PER_CLASS
    k_row = t2i % _MAX_PER_CLASS
    sub80 = jax.lax.broadcasted_iota(jnp.int32, (_NUM_CLASSES, 128), 0)
    sub100 = jax.lax.broadcasted_iota(jnp.int32, (_MAX_PER_CLASS, 128), 0)
    oct_ = (sub80 == c_row).astype(jnp.float32)
    okt = (sub100 == k_row).astype(jnp.float32)

    def pick(coord):
        p = jax.lax.dot(coord, okt, precision=jax.lax.Precision.HIGHEST)
        return jnp.sum(oct_ * p, axis=0, keepdims=True)

    bx1 = jnp.sum(eye * pick(x1), axis=1, keepdims=True)
    by1 = jnp.sum(eye * pick(y1), axis=1, keepdims=True)
    bx2 = jnp.sum(eye * pick(x2), axis=1, keepdims=True)
    by2 = jnp.sum(eye * pick(y2), axis=1, keepdims=True)
    fc_col = (fi_col // _MAX_PER_CLASS).astype(jnp.float32)

    out = jnp.concatenate([bx1, by1, bx2, by2, fs_col, fc_col], axis=1)
    out_ref[0] = out[:_MAX_PER_CLASS, :]


def _nms_finalize(tv, bx1, by1, bx2, by2):
    n_img = tv.shape[0]
    spec_s = pl.BlockSpec((1, _NUM_CLASSES, 128), lambda i: (i, 0, 0))
    spec_b = pl.BlockSpec((1, _NUM_CLASSES, _MAX_PER_CLASS), lambda i: (i, 0, 0))
    return pl.pallas_call(
        _nms_body,
        grid=(n_img,),
        in_specs=[spec_s, spec_b, spec_b, spec_b, spec_b],
        out_specs=pl.BlockSpec((1, _MAX_PER_CLASS, 6), lambda i: (i, 0, 0)),
        out_shape=jax.ShapeDtypeStruct((n_img, _MAX_PER_CLASS, 6), jnp.float32),
        scratch_shapes=[
            pltpu.VMEM((_NUM_CLASSES, _MAX_PER_CLASS, _MAX_PER_CLASS),
                       jnp.float32),
            pltpu.VMEM((_NUM_CLASSES, _MAX_PER_CLASS), jnp.float32),
        ],
    )(tv, bx1, by1, bx2, by2)


def kernel(images, predictions):
    n_img = predictions.shape[0]
    anchors = jnp.asarray(_anchors_np(images.shape[1], images.shape[2]))

    b = predictions[..., :4] * jnp.asarray(_BOX_VAR)
    cxcy = b[..., :2] * anchors[:, 2:] + anchors[:, :2]
    wh = jnp.exp(b[..., 2:]) * anchors[:, 2:]
    boxes = jnp.concatenate([cxcy - 0.5 * wh, cxcy + 0.5 * wh], axis=-1)
    scores = jax.nn.sigmoid(predictions[..., 4:])

    sp = jnp.pad(scores, ((0, 0), (0, _N_PAD - _N_ANCHORS), (0, 0)),
                 constant_values=_NEG)
    sct = sp.reshape(n_img, _NUM_CHUNKS, 128, _NUM_CLASSES).transpose(0, 3, 1, 2)

    m = _chunk_maxes(sct).reshape(n_img, _NUM_CLASSES, _NUM_CHUNKS // 128, 128)
    m2 = m.transpose(0, 2, 1, 3)
    _, qi = _per_class_topk(m2)
    cids = qi[:, :, :_MAX_PER_CLASS]

    class_off = (jnp.arange(n_img, dtype=jnp.int32)[:, None, None] * _NUM_CLASSES
                 + jnp.arange(_NUM_CLASSES, dtype=jnp.int32)[None, :, None])
    row_idx = (class_off * _NUM_CHUNKS + cids).reshape(
        1, n_img * _NUM_CLASSES * _MAX_PER_CLASS)
    cand_rows = _gather_rows_sc(
        sct.reshape(n_img * _NUM_CLASSES * _NUM_CHUNKS, 128), row_idx)
    cands = cand_rows.reshape(
        n_img, _NUM_CLASSES, _MAX_PER_CLASS, 128).transpose(0, 2, 1, 3)
    cids_b = cids.transpose(0, 2, 1)[..., None]

    tv, ti = _cand_topk(cands, cids_b)

    idx100 = ti[:, :, :_MAX_PER_CLASS]
    img_off = (jnp.arange(n_img, dtype=jnp.int32) * _N_ANCHORS)[:, None, None]
    flat_idx = (idx100 + img_off).reshape(1, n_img * _NUM_CLASSES * _MAX_PER_CLASS)
    table = jnp.pad(boxes.reshape(n_img * _N_ANCHORS, 4), ((0, 0), (0, 124)))
    g = _gather_rows_sc(table, flat_idx)
    gb = g[:, :4].reshape(n_img, _NUM_CLASSES, _MAX_PER_CLASS, 4)

    return _nms_finalize(tv, gb[..., 0], gb[..., 1], gb[..., 2], gb[..., 3])
